# baseline (speedup 1.0000x reference)
import functools

import jax
import jax.numpy as jnp
from jax.experimental import pallas as pl
from jax.experimental.pallas import tpu as pltpu

T = 256
D_IN = 2048
D_OUT = 2048
E = 8
TN = 512
N_TILES = D_OUT // TN


def _dense_body(x_ref, gw_ref, gb_ref, w_ref, b_ref, out_ref, aux_ref, oh_ref):
    n = pl.program_id(0)
    e = pl.program_id(1)

    @pl.when((n == 0) & (e == 0))
    def _gate():
        logits = jax.lax.dot_general(
            x_ref[...], gw_ref[...], (((1,), (1,)), ((), ())),
            preferred_element_type=jnp.float32,
        ) + gb_ref[...]
        m = jnp.max(logits, axis=1, keepdims=True)
        unn = jnp.exp(logits - m)
        p = unn / jnp.sum(unn, axis=1, keepdims=True)
        lane = jax.lax.broadcasted_iota(jnp.int32, (T, E), 1)
        is_max = p == jnp.max(p, axis=1, keepdims=True)
        top1 = jnp.min(jnp.where(is_max, lane, E), axis=1, keepdims=True)
        oh_ref[...] = (lane == top1).astype(jnp.float32)
        mean_gate = jnp.mean(p, axis=0, keepdims=True)
        aux_ref[...] = jnp.mean((mean_gate * E) ** 2, axis=1, keepdims=True)

    acc = jax.lax.dot_general(
        x_ref[...], w_ref[0], (((1,), (1,)), ((), ())),
        preferred_element_type=jnp.float32,
    ) + b_ref[0]
    lane = jax.lax.broadcasted_iota(jnp.int32, (T, E), 1)
    mask = jnp.sum(jnp.where(lane == e, oh_ref[...], 0.0), axis=1,
                   keepdims=True)

    @pl.when(e == 0)
    def _init():
        out_ref[...] = mask * acc

    @pl.when(e > 0)
    def _accum():
        out_ref[...] += mask * acc


def kernel(x, gate_W, gate_b, W, b):
    out, aux = pl.pallas_call(
        _dense_body,
        grid=(N_TILES, E),
        in_specs=[
            pl.BlockSpec((T, D_IN), lambda n, e: (0, 0)),
            pl.BlockSpec((E, D_IN), lambda n, e: (0, 0)),
            pl.BlockSpec((1, E), lambda n, e: (0, 0)),
            pl.BlockSpec((1, TN, D_IN), lambda n, e: (e, n, 0)),
            pl.BlockSpec((1, 1, TN), lambda n, e: (e, 0, n)),
        ],
        out_specs=[
            pl.BlockSpec((T, TN), lambda n, e: (0, n)),
            pl.BlockSpec((1, 1), lambda n, e: (0, 0)),
        ],
        out_shape=[
            jax.ShapeDtypeStruct((T, D_OUT), jnp.float32),
            jax.ShapeDtypeStruct((1, 1), jnp.float32),
        ],
        scratch_shapes=[pltpu.VMEM((T, E), jnp.float32)],
    )(x, gate_W, gate_b.reshape(1, E), W, b.reshape(E, 1, D_OUT))
    return out, aux[0, 0]
